# baseline (speedup 1.0000x reference)
import functools

import jax
import jax.numpy as jnp
from jax import lax
from jax.experimental import pallas as pl
from jax.experimental.pallas import tpu as pltpu
from jax.experimental.pallas import tpu_sc as plsc

NUM_CORES = 2
NUM_SUBCORES = 16
NUM_WORKERS = NUM_CORES * NUM_SUBCORES

CHUNK = 8
NBUF = 4
LANES = 16

TC_FRACTION = 0.75
TC_BLK = 256


def _sc_body(sc_rows, row0, d, x_hbm, pos_hbm, table_hbm, out_hbm,
             idx_v, buf_v, rows_v, sem_in, sem_out):
    rows_per_w = sc_rows // NUM_WORKERS
    n_chunks = rows_per_w // CHUNK
    d_vecs = d // LANES
    wid = lax.axis_index("s") * NUM_CORES + lax.axis_index("c")
    base = row0 + wid * rows_per_w

    pltpu.sync_copy(pos_hbm.at[pl.ds(base, rows_per_w)], idx_v)

    def start_loads(g, b):
        row = pl.ds(base + g * CHUNK, CHUNK)
        idx = idx_v.at[pl.ds(g * CHUNK, CHUNK)]
        cp_t = pltpu.async_copy(table_hbm.at[idx], rows_v.at[b], sem_in.at[b])
        cp_x = pltpu.async_copy(x_hbm.at[row], buf_v.at[b], sem_in.at[b])
        return cp_t, cp_x

    PREFETCH = NBUF - 2

    def process(g, b):
        nxt = g + PREFETCH
        target = (b + PREFETCH) % NBUF

        @pl.when(nxt >= NBUF)
        def _():
            pltpu.make_async_copy(
                buf_v.at[target], out_hbm.at[pl.ds(base, CHUNK)],
                sem_out.at[target]).wait()

        @pl.when(nxt < n_chunks)
        def _():
            start_loads(nxt, target)

        row = pl.ds(base + g * CHUNK, CHUNK)
        idx = idx_v.at[pl.ds(g * CHUNK, CHUNK)]
        pltpu.make_async_copy(x_hbm.at[row], buf_v.at[b], sem_in.at[b]).wait()
        pltpu.make_async_copy(table_hbm.at[idx], rows_v.at[b],
                              sem_in.at[b]).wait()

        buf = buf_v.at[b]
        rows = rows_v.at[b]

        @plsc.parallel_loop(0, CHUNK * d_vecs, unroll=8)
        def _(i):
            r = i // d_vecs
            sl = pl.ds((i % d_vecs) * LANES, LANES)
            buf[r, sl] = buf[r, sl] + rows[r, sl]

        pltpu.async_copy(buf_v.at[b], out_hbm.at[row], sem_out.at[b])

    def super_step(gg, carry):
        for b in range(NBUF):
            process(gg * NBUF + b, b)
        return carry

    for g in range(PREFETCH):
        start_loads(g, g % NBUF)
    lax.fori_loop(0, n_chunks // NBUF, super_step, 0)

    for g in range(n_chunks - PREFETCH, n_chunks):
        b_last = g % NBUF
        pltpu.make_async_copy(
            buf_v.at[b_last], out_hbm.at[pl.ds(base, CHUNK)],
            sem_out.at[b_last]).wait()


def _tc_body(pos_ref, x_ref, table_ref, prev_ref, out_ref):
    del prev_ref

    def row(r, carry):
        p = pos_ref[0, 0, r]
        out_ref[r] = x_ref[r] + table_ref[p]
        return carry

    lax.fori_loop(0, TC_BLK, row, 0, unroll=8)


def kernel(x, x_pos, table):
    b, s, d = x.shape
    n_rows = b * s
    sub = d // 128
    x2d = x.reshape(n_rows, d)
    pos = x_pos.reshape(n_rows).astype(jnp.int32)

    tc_rows = int(n_rows * TC_FRACTION)
    sc_rows = n_rows - tc_rows

    mesh = plsc.VectorSubcoreMesh(
        core_axis_name="c", subcore_axis_name="s",
        num_cores=NUM_CORES, num_subcores=NUM_SUBCORES)
    sc_body = functools.partial(_sc_body, sc_rows, tc_rows, d)
    sc_out = pl.kernel(
        sc_body,
        out_type=jax.ShapeDtypeStruct((n_rows, d), jnp.float32),
        mesh=mesh,
        scratch_types=[
            pltpu.VMEM((sc_rows // NUM_WORKERS,), jnp.int32),
            pltpu.VMEM((NBUF, CHUNK, d), jnp.float32),
            pltpu.VMEM((NBUF, CHUNK, d), jnp.float32),
            pltpu.SemaphoreType.DMA((NBUF,)),
            pltpu.SemaphoreType.DMA((NBUF,)),
        ],
    )(x2d, pos, table)

    n_blocks = tc_rows // TC_BLK
    x3 = x2d.reshape(n_rows, sub, 128)
    table3 = table.reshape(table.shape[0], sub, 128)
    pos3 = pos.reshape(n_rows // TC_BLK, 1, TC_BLK)
    sc_out3 = sc_out.reshape(n_rows, sub, 128)

    out3 = pl.pallas_call(
        _tc_body,
        grid=(n_blocks,),
        in_specs=[
            pl.BlockSpec((1, 1, TC_BLK), lambda i: (i, 0, 0),
                         memory_space=pltpu.SMEM),
            pl.BlockSpec((TC_BLK, sub, 128), lambda i: (i, 0, 0)),
            pl.BlockSpec((table.shape[0], sub, 128), lambda i: (0, 0, 0)),
            pl.BlockSpec(memory_space=pl.ANY),
        ],
        out_specs=pl.BlockSpec((TC_BLK, sub, 128), lambda i: (i, 0, 0)),
        out_shape=jax.ShapeDtypeStruct((n_rows, sub, 128), jnp.float32),
        input_output_aliases={3: 0},
    )(pos3, x3, table3, sc_out3)

    return out3.reshape(b, s, d)


# speedup vs baseline: 3.3642x; 3.3642x over previous
import functools

import jax
import jax.numpy as jnp
from jax import lax
from jax.experimental import pallas as pl
from jax.experimental.pallas import tpu as pltpu
from jax.experimental.pallas import tpu_sc as plsc

NUM_CORES = 2
NUM_SUBCORES = 16
NUM_WORKERS = NUM_CORES * NUM_SUBCORES

CHUNK = 8
NBUF = 4
LANES = 16


def _pos_embed_body(n_rows, d, x_hbm, pos_hbm, table_hbm, out_hbm,
                    idx_v, buf_v, rows_v, sem_in, sem_out):
    rows_per_w = n_rows // NUM_WORKERS
    n_chunks = rows_per_w // CHUNK
    d_vecs = d // LANES
    wid = lax.axis_index("s") * NUM_CORES + lax.axis_index("c")
    base = wid * rows_per_w

    pltpu.sync_copy(pos_hbm.at[pl.ds(base, rows_per_w)], idx_v)

    def start_loads(g, b):
        row = pl.ds(base + g * CHUNK, CHUNK)
        idx = idx_v.at[pl.ds(g * CHUNK, CHUNK)]
        cp_x = pltpu.async_copy(x_hbm.at[row], buf_v.at[b], sem_in.at[b])
        cp_t = pltpu.async_copy(table_hbm.at[idx], rows_v.at[b], sem_in.at[b])
        return cp_x, cp_t

    PREFETCH = NBUF - 2

    def process(g, b):
        nxt = g + PREFETCH
        target = (b + PREFETCH) % NBUF

        @pl.when(nxt >= NBUF)
        def _():
            pltpu.make_async_copy(
                buf_v.at[target], out_hbm.at[pl.ds(base, CHUNK)],
                sem_out.at[target]).wait()

        @pl.when(nxt < n_chunks)
        def _():
            start_loads(nxt, target)

        row = pl.ds(base + g * CHUNK, CHUNK)
        idx = idx_v.at[pl.ds(g * CHUNK, CHUNK)]
        pltpu.make_async_copy(x_hbm.at[row], buf_v.at[b], sem_in.at[b]).wait()
        pltpu.make_async_copy(table_hbm.at[idx], rows_v.at[b],
                              sem_in.at[b]).wait()

        buf = buf_v.at[b]
        rows = rows_v.at[b]

        @plsc.parallel_loop(0, CHUNK * d_vecs, unroll=8)
        def _(i):
            r = i // d_vecs
            sl = pl.ds((i % d_vecs) * LANES, LANES)
            buf[r, sl] = buf[r, sl] + rows[r, sl]

        pltpu.async_copy(buf_v.at[b], out_hbm.at[row], sem_out.at[b])

    def super_step(gg, carry):
        for b in range(NBUF):
            process(gg * NBUF + b, b)
        return carry

    for g in range(PREFETCH):
        start_loads(g, g % NBUF)
    lax.fori_loop(0, n_chunks // NBUF, super_step, 0)

    for g in range(n_chunks - PREFETCH, n_chunks):
        b_last = g % NBUF
        pltpu.make_async_copy(
            buf_v.at[b_last], out_hbm.at[pl.ds(base, CHUNK)],
            sem_out.at[b_last]).wait()


def kernel(x, x_pos, table):
    b, s, d = x.shape
    n_rows = b * s
    x2d = x.reshape(n_rows, d)
    pos = x_pos.reshape(n_rows).astype(jnp.int32)

    mesh = plsc.VectorSubcoreMesh(
        core_axis_name="c", subcore_axis_name="s",
        num_cores=NUM_CORES, num_subcores=NUM_SUBCORES)

    rows_per_w = n_rows // NUM_WORKERS
    body = functools.partial(_pos_embed_body, n_rows, d)
    out2d = pl.kernel(
        body,
        out_type=jax.ShapeDtypeStruct((n_rows, d), jnp.float32),
        mesh=mesh,
        scratch_types=[
            pltpu.VMEM((rows_per_w,), jnp.int32),
            pltpu.VMEM((NBUF, CHUNK, d), jnp.float32),
            pltpu.VMEM((NBUF, CHUNK, d), jnp.float32),
            pltpu.SemaphoreType.DMA((NBUF,)),
            pltpu.SemaphoreType.DMA((NBUF,)),
        ],
    )(x2d, pos, table)
    return out2d.reshape(b, s, d)
